# baseline (speedup 1.0000x reference)
import functools

import jax
import jax.numpy as jnp
from jax import lax
from jax.experimental import pallas as pl
from jax.experimental.pallas import tpu as pltpu, tpu_sc as plsc

N = 10000
NACC = 10240
E = 320000
D = 128
SUB = 16
CORES = 2
CH = 32
Q = 24
NSC = 14
NCH = Q * NSC
RING = 6
EPS = NCH * CH
EPAD = CORES * SUB * EPS
RPT = NACC // SUB


def _mm1_body(x_ref, w_ref, b_ref, o_ref):
    o_ref[...] = (
        jnp.dot(x_ref[...], w_ref[...], preferred_element_type=jnp.float32)
        + b_ref[...]
    )


def _matmul1(x, W, b):
    BM = 400
    return pl.pallas_call(
        _mm1_body,
        grid=(N // BM,),
        in_specs=[
            pl.BlockSpec((BM, D), lambda i: (i, 0)),
            pl.BlockSpec((D, D), lambda i: (0, 0)),
            pl.BlockSpec((1, D), lambda i: (0, 0)),
        ],
        out_specs=pl.BlockSpec((BM, D), lambda i: (i, 0)),
        out_shape=jax.ShapeDtypeStruct((N, D), jnp.float32),
    )(x, W, b)


def _mm2_body(a_ref, b_ref, wa_ref, wb_ref, bias_ref, o_ref):
    x = a_ref[0] + b_ref[0]
    o_ref[...] = (
        jnp.dot(jnp.maximum(x, 0.0), wa_ref[...],
                preferred_element_type=jnp.float32)
        + jnp.dot(jnp.minimum(x, 0.0), wb_ref[...],
                  preferred_element_type=jnp.float32)
        + bias_ref[...]
    )


def _matmul2(parts, Wa, Wb, b):
    BM = 400
    return pl.pallas_call(
        _mm2_body,
        grid=(N // BM,),
        in_specs=[
            pl.BlockSpec((1, BM, D), lambda i: (0, i, 0)),
            pl.BlockSpec((1, BM, D), lambda i: (1, i, 0)),
            pl.BlockSpec((D, D), lambda i: (0, 0)),
            pl.BlockSpec((D, D), lambda i: (0, 0)),
            pl.BlockSpec((1, D), lambda i: (0, 0)),
        ],
        out_specs=pl.BlockSpec((BM, D), lambda i: (i, 0)),
        out_shape=jax.ShapeDtypeStruct((N, D), jnp.float32),
    )(parts, parts, Wa, Wb, b)


def _conv_body(h_hbm, src_hbm, dst_hbm, w_hbm, out_hbm,
               src_i, dst_i, w_i, *rest):
    rows = rest[:RING]
    accum = rest[RING]
    gsems = rest[RING + 1:2 * RING + 1]
    ssems = rest[2 * RING + 1:3 * RING + 1]
    isems = rest[3 * RING + 1:3 * RING + 3]
    c = lax.axis_index("c")
    s = lax.axis_index("s")

    zero = jnp.zeros((16,), jnp.float32)

    def zb(i, carry):
        rows[0][i // 8, pl.ds((i % 8) * 16, 16)] = zero
        return carry

    lax.fori_loop(0, CH * 8, zb, 0)
    r0 = s * RPT
    for k in range(RPT // CH):
        pltpu.sync_copy(rows[0], accum.at[pl.ds(r0 + CH * k, CH), :])

    pltpu.sync_copy(src_hbm.at[c, s, 0], src_i.at[0])
    pltpu.sync_copy(dst_hbm.at[c, s, 0], dst_i.at[0])
    pltpu.sync_copy(w_hbm.at[c, s, 0], w_i.at[0])
    plsc.subcore_barrier()

    def _fetch_idx(slot, u):
        pltpu.async_copy(src_hbm.at[c, s, u], src_i.at[slot], isems[0])
        pltpu.async_copy(dst_hbm.at[c, s, u], dst_i.at[slot], isems[0])
        pltpu.async_copy(w_hbm.at[c, s, u], w_i.at[slot], isems[1])

    def _wait_idx(slot, u):
        pltpu.make_async_copy(src_hbm.at[c, s, u], src_i.at[slot],
                              isems[0]).wait()
        pltpu.make_async_copy(dst_hbm.at[c, s, u], dst_i.at[slot],
                              isems[0]).wait()
        pltpu.make_async_copy(w_hbm.at[c, s, u], w_i.at[slot],
                              isems[1]).wait()

    for b in range(RING - 2):
        pltpu.async_copy(h_hbm.at[src_i.at[0, b]], rows[b], gsems[b])

    def _scale(buf, isl, k):
        def grp(g, carry2):
            base = g * 16
            wrow = w_i[isl, k, pl.ds(base, 16)]
            for e in range(16):
                wv = jnp.full((16,), wrow[e])
                for f in range(8):
                    sl = (base + e, pl.ds(16 * f, 16))
                    buf[sl] = buf[sl] * wv
            return carry2

        lax.fori_loop(0, CH // 16, grp, 0)

    def super_body(u, carry):
        isl = lax.rem(u, 2)
        inxt = lax.rem(u + 1, 2)

        def inner_ring(kk, carry2):
            for p in range(RING):
                k = RING * kk + p
                j = Q * u + k
                pltpu.make_async_copy(
                    h_hbm.at[src_i.at[isl, k]], rows[p], gsems[p]
                ).wait()

                _scale(rows[p], isl, k)
                pltpu.async_copy(
                    rows[p], accum.at[dst_i.at[isl, k]], ssems[p], add=True
                )

                @pl.when(j >= 2)
                def _():
                    km = k - 2
                    slp = jnp.where(km >= 0, isl, inxt)
                    kpr = lax.rem(km + Q, Q)
                    pltpu.make_async_copy(
                        rows[(p - 2) % RING],
                        accum.at[dst_i.at[slp, kpr]],
                        ssems[(p - 2) % RING],
                    ).wait()

                if p == 2:
                    @pl.when((kk == 0) & (u + 1 < NSC))
                    def _():
                        _fetch_idx(inxt, u + 1)

                    @pl.when((kk == Q // RING - 1) & (u + 1 < NSC))
                    def _():
                        _wait_idx(inxt, u + 1)

                @pl.when(j + RING - 2 < NCH)
                def _():
                    kn = k + RING - 2
                    sln = jnp.where(kn < Q, isl, inxt)
                    knr = lax.rem(kn, Q)
                    pltpu.async_copy(
                        h_hbm.at[src_i.at[sln, knr]],
                        rows[(p - 2) % RING],
                        gsems[(p - 2) % RING],
                    )
            return carry2

        lax.fori_loop(0, Q // RING, inner_ring, 0)
        return carry

    lax.fori_loop(0, NSC, super_body, 0)
    lsl = (NSC - 1) % 2
    for k in range(Q - 2, Q):
        pltpu.make_async_copy(
            rows[k % RING], accum.at[dst_i.at[lsl, k]], ssems[k % RING]
        ).wait()
    plsc.subcore_barrier()

    @pl.when(s < SUB - 1)
    def _():
        pltpu.sync_copy(
            accum.at[pl.ds(r0, RPT), :],
            out_hbm.at[c, pl.ds(r0, RPT), :],
        )

    @pl.when(s == SUB - 1)
    def _():
        last = N - (SUB - 1) * RPT
        pltpu.sync_copy(
            accum.at[pl.ds((SUB - 1) * RPT, last), :],
            out_hbm.at[c, pl.ds((SUB - 1) * RPT, last), :],
        )


_conv = functools.partial(
    pl.kernel,
    out_type=jax.ShapeDtypeStruct((CORES, N, D), jnp.float32),
    mesh=plsc.VectorSubcoreMesh(core_axis_name="c", subcore_axis_name="s"),
    scratch_types=(
        [
            pltpu.VMEM((2, Q, CH), jnp.int32),
            pltpu.VMEM((2, Q, CH), jnp.int32),
            pltpu.VMEM((2, Q, CH), jnp.float32),
        ]
        + [pltpu.VMEM((CH, D), jnp.float32) for _ in range(RING)]
        + [pltpu.VMEM_SHARED((NACC, D), jnp.float32)]
        + [pltpu.SemaphoreType.DMA for _ in range(2 * RING + 2)]
    ),
)(_conv_body)


def _prep_indices(edge_index, edge_weight):
    src = edge_index[0].astype(jnp.int32)
    dst = edge_index[1].astype(jnp.int32)
    w = edge_weight.astype(jnp.float32)
    pad = EPAD - E
    src_g = jnp.pad(src, (0, pad)).reshape(CORES, SUB, NSC, Q, CH)
    dst_g = jnp.pad(dst, (0, pad)).reshape(CORES, SUB, NSC, Q, CH)
    w_g = jnp.pad(w, (0, pad)).reshape(CORES, SUB, NSC, Q, CH)
    return src_g, dst_g, w_g


def kernel(x, edge_index, edge_weight, W1, b1, W2, b2):
    src_g, dst_g, w_g = _prep_indices(edge_index, edge_weight)
    b1r = b1.reshape(1, D)
    b2r = b2.reshape(1, D)

    h = _matmul1(x, W1, b1r)

    eye = jnp.eye(D, dtype=jnp.float32)
    Was = jnp.stack([W2, eye])
    Wbs = jnp.stack([jnp.zeros((D, D), jnp.float32), eye])
    bs = jnp.stack([b2r, jnp.zeros((1, D), jnp.float32)])

    def body(carry, xs):
        Wa, Wb, b = xs
        parts = _conv(carry, src_g, dst_g, w_g)
        return _matmul2(parts, Wa, Wb, b), None

    h, _ = lax.scan(body, h, (Was, Wbs, bs))
    return h


# speedup vs baseline: 2.1563x; 2.1563x over previous
import functools

import jax
import jax.numpy as jnp
from jax import lax
from jax.experimental import pallas as pl
from jax.experimental.pallas import tpu as pltpu, tpu_sc as plsc

N = 10000
NACC = 10240
E = 320000
D = 128
SUB = 16
CORES = 2
CH = 128
Q = 8
NSC = 10
NCH = Q * NSC
EPS = NCH * CH
EPAD = CORES * SUB * EPS
RPT = NACC // SUB


def _mm1_body(x_ref, w_ref, b_ref, o_ref):
    o_ref[...] = (
        jnp.dot(x_ref[...], w_ref[...], preferred_element_type=jnp.float32)
        + b_ref[...]
    )


def _matmul1(x, W, b):
    BM = 400
    return pl.pallas_call(
        _mm1_body,
        grid=(N // BM,),
        in_specs=[
            pl.BlockSpec((BM, D), lambda i: (i, 0)),
            pl.BlockSpec((D, D), lambda i: (0, 0)),
            pl.BlockSpec((1, D), lambda i: (0, 0)),
        ],
        out_specs=pl.BlockSpec((BM, D), lambda i: (i, 0)),
        out_shape=jax.ShapeDtypeStruct((N, D), jnp.float32),
    )(x, W, b)


def _mm2_body(a_ref, b_ref, wa_ref, wb_ref, bias_ref, o_ref):
    x = a_ref[0] + b_ref[0]
    o_ref[...] = (
        jnp.dot(jnp.maximum(x, 0.0), wa_ref[...],
                preferred_element_type=jnp.float32)
        + jnp.dot(jnp.minimum(x, 0.0), wb_ref[...],
                  preferred_element_type=jnp.float32)
        + bias_ref[...]
    )


def _matmul2(parts, Wa, Wb, b):
    BM = 400
    return pl.pallas_call(
        _mm2_body,
        grid=(N // BM,),
        in_specs=[
            pl.BlockSpec((1, BM, D), lambda i: (0, i, 0)),
            pl.BlockSpec((1, BM, D), lambda i: (1, i, 0)),
            pl.BlockSpec((D, D), lambda i: (0, 0)),
            pl.BlockSpec((D, D), lambda i: (0, 0)),
            pl.BlockSpec((1, D), lambda i: (0, 0)),
        ],
        out_specs=pl.BlockSpec((BM, D), lambda i: (i, 0)),
        out_shape=jax.ShapeDtypeStruct((N, D), jnp.float32),
    )(parts, parts, Wa, Wb, b)


def _conv_body(h_hbm, src_hbm, dst_hbm, w_hbm, out_hbm,
               src_i, dst_i, w_i, rows0, rows1, accum,
               gsem0, gsem1, isem0, isem1):
    rows = (rows0, rows1)
    gsems = (gsem0, gsem1)
    isems = (isem0, isem1)
    c = lax.axis_index("c")
    s = lax.axis_index("s")

    zero = jnp.zeros((16,), jnp.float32)

    def zb(i, carry):
        rows[0][i // 8, pl.ds((i % 8) * 16, 16)] = zero
        return carry

    lax.fori_loop(0, CH * 8, zb, 0)
    r0 = s * RPT
    for k in range(RPT // CH):
        pltpu.sync_copy(rows[0], accum.at[pl.ds(r0 + CH * k, CH), :])

    pltpu.sync_copy(src_hbm.at[c, s, 0], src_i.at[0])
    pltpu.sync_copy(dst_hbm.at[c, s, 0], dst_i.at[0])
    pltpu.sync_copy(w_hbm.at[c, s, 0], w_i.at[0])
    plsc.subcore_barrier()

    def _fetch_idx(slot, u):
        pltpu.async_copy(src_hbm.at[c, s, u], src_i.at[slot], isems[0])
        pltpu.async_copy(dst_hbm.at[c, s, u], dst_i.at[slot], isems[0])
        pltpu.async_copy(w_hbm.at[c, s, u], w_i.at[slot], isems[1])

    def _wait_idx(slot, u):
        pltpu.make_async_copy(src_hbm.at[c, s, u], src_i.at[slot],
                              isems[0]).wait()
        pltpu.make_async_copy(dst_hbm.at[c, s, u], dst_i.at[slot],
                              isems[0]).wait()
        pltpu.make_async_copy(w_hbm.at[c, s, u], w_i.at[slot],
                              isems[1]).wait()

    pltpu.async_copy(h_hbm.at[src_i.at[0, 0]], rows[0], gsems[0])

    def _scale(buf, isl, k):
        def grp(g, carry2):
            base = g * 16
            wrow = w_i[isl, k, pl.ds(base, 16)]
            for e in range(16):
                wv = jnp.full((16,), wrow[e])
                for f in range(8):
                    sl = (base + e, pl.ds(16 * f, 16))
                    buf[sl] = buf[sl] * wv
            return carry2

        lax.fori_loop(0, CH // 16, grp, 0)

    def super_body(u, carry):
        isl = lax.rem(u, 2)
        inxt = lax.rem(u + 1, 2)

        def pair(kk, carry2):
            for p in range(2):
                k = 2 * kk + p
                j = Q * u + k
                pltpu.make_async_copy(
                    h_hbm.at[src_i.at[isl, k]], rows[p], gsems[p]
                ).wait()

                if p == 0:
                    @pl.when((kk == 1) & (u + 1 < NSC))
                    def _():
                        _fetch_idx(inxt, u + 1)
                else:
                    @pl.when((kk == Q // 2 - 1) & (u + 1 < NSC))
                    def _():
                        _wait_idx(inxt, u + 1)

                _scale(rows[p], isl, k)
                pltpu.sync_copy(rows[p], accum.at[dst_i.at[isl, k]],
                                add=True)

                @pl.when(j + 1 < NCH)
                def _():
                    kn = k + 1
                    sln = jnp.where(kn < Q, isl, inxt)
                    knr = lax.rem(kn, Q)
                    pltpu.async_copy(
                        h_hbm.at[src_i.at[sln, knr]],
                        rows[1 - p],
                        gsems[1 - p],
                    )
            return carry2

        lax.fori_loop(0, Q // 2, pair, 0)
        return carry

    lax.fori_loop(0, NSC, super_body, 0)
    plsc.subcore_barrier()

    @pl.when(s < SUB - 1)
    def _():
        pltpu.sync_copy(
            accum.at[pl.ds(r0, RPT), :],
            out_hbm.at[c, pl.ds(r0, RPT), :],
        )

    @pl.when(s == SUB - 1)
    def _():
        last = N - (SUB - 1) * RPT
        pltpu.sync_copy(
            accum.at[pl.ds((SUB - 1) * RPT, last), :],
            out_hbm.at[c, pl.ds((SUB - 1) * RPT, last), :],
        )


_conv = functools.partial(
    pl.kernel,
    out_type=jax.ShapeDtypeStruct((CORES, N, D), jnp.float32),
    mesh=plsc.VectorSubcoreMesh(core_axis_name="c", subcore_axis_name="s"),
    scratch_types=(
        [
            pltpu.VMEM((2, Q, CH), jnp.int32),
            pltpu.VMEM((2, Q, CH), jnp.int32),
            pltpu.VMEM((2, Q, CH), jnp.float32),
        ]
        + [pltpu.VMEM((CH, D), jnp.float32) for _ in range(2)]
        + [pltpu.VMEM_SHARED((NACC, D), jnp.float32)]
        + [pltpu.SemaphoreType.DMA for _ in range(4)]
    ),
)(_conv_body)


def _prep_indices(edge_index, edge_weight):
    src = edge_index[0].astype(jnp.int32)
    dst = edge_index[1].astype(jnp.int32)
    w = edge_weight.astype(jnp.float32)
    pad = EPAD - E
    src_g = jnp.pad(src, (0, pad)).reshape(CORES, SUB, NSC, Q, CH)
    dst_g = jnp.pad(dst, (0, pad)).reshape(CORES, SUB, NSC, Q, CH)
    w_g = jnp.pad(w, (0, pad)).reshape(CORES, SUB, NSC, Q, CH)
    return src_g, dst_g, w_g


def kernel(x, edge_index, edge_weight, W1, b1, W2, b2):
    src_g, dst_g, w_g = _prep_indices(edge_index, edge_weight)
    b1r = b1.reshape(1, D)
    b2r = b2.reshape(1, D)

    h = _matmul1(x, W1, b1r)

    eye = jnp.eye(D, dtype=jnp.float32)
    Was = jnp.stack([W2, eye])
    Wbs = jnp.stack([jnp.zeros((D, D), jnp.float32), eye])
    bs = jnp.stack([b2r, jnp.zeros((1, D), jnp.float32)])

    def body(carry, xs):
        Wa, Wb, b = xs
        parts = _conv(carry, src_g, dst_g, w_g)
        return _matmul2(parts, Wa, Wb, b), None

    h, _ = lax.scan(body, h, (Was, Wbs, bs))
    return h


# speedup vs baseline: 2.9443x; 1.3654x over previous
import functools

import jax
import jax.numpy as jnp
from jax import lax
from jax.experimental import pallas as pl
from jax.experimental.pallas import tpu as pltpu, tpu_sc as plsc

N = 10000
NACC = 10240
E = 320000
D = 128
SUB = 16
CORES = 2
CH = 128
NCH = 79
EPS = NCH * CH
EPAD = CORES * SUB * EPS
RPT = NACC // SUB


def _mm1_body(x_ref, w_ref, b_ref, o_ref):
    o_ref[...] = (
        jnp.dot(x_ref[...], w_ref[...], preferred_element_type=jnp.float32)
        + b_ref[...]
    )


def _matmul1(x, W, b):
    BM = 400
    return pl.pallas_call(
        _mm1_body,
        grid=(N // BM,),
        in_specs=[
            pl.BlockSpec((BM, D), lambda i: (i, 0)),
            pl.BlockSpec((D, D), lambda i: (0, 0)),
            pl.BlockSpec((1, D), lambda i: (0, 0)),
        ],
        out_specs=pl.BlockSpec((BM, D), lambda i: (i, 0)),
        out_shape=jax.ShapeDtypeStruct((N, D), jnp.float32),
    )(x, W, b)


def _mm2_body(a_ref, b_ref, w_ref, bias_ref, o_ref):
    x = jnp.maximum(a_ref[0] + b_ref[0], 0.0)
    o_ref[...] = (
        jnp.dot(x, w_ref[...], preferred_element_type=jnp.float32)
        + bias_ref[...]
    )


def _matmul2(parts, W, b):
    BM = 400
    return pl.pallas_call(
        _mm2_body,
        grid=(N // BM,),
        in_specs=[
            pl.BlockSpec((1, BM, D), lambda i: (0, i, 0)),
            pl.BlockSpec((1, BM, D), lambda i: (1, i, 0)),
            pl.BlockSpec((D, D), lambda i: (0, 0)),
            pl.BlockSpec((1, D), lambda i: (0, 0)),
        ],
        out_specs=pl.BlockSpec((BM, D), lambda i: (i, 0)),
        out_shape=jax.ShapeDtypeStruct((N, D), jnp.float32),
    )(parts, parts, W, b)


def _add_body(a_ref, b_ref, o_ref):
    o_ref[...] = a_ref[0] + b_ref[0]


def _add_parts(parts):
    BM = 400
    return pl.pallas_call(
        _add_body,
        grid=(N // BM,),
        in_specs=[
            pl.BlockSpec((1, BM, D), lambda i: (0, i, 0)),
            pl.BlockSpec((1, BM, D), lambda i: (1, i, 0)),
        ],
        out_specs=pl.BlockSpec((BM, D), lambda i: (i, 0)),
        out_shape=jax.ShapeDtypeStruct((N, D), jnp.float32),
    )(parts, parts)


def _conv_body(h_hbm, src_hbm, dst_hbm, w_hbm, out_hbm,
               src_v, dst_v, w_v, rows_v, accum, sem):
    c = lax.axis_index("c")
    s = lax.axis_index("s")

    pltpu.sync_copy(src_hbm.at[c, s], src_v)
    pltpu.sync_copy(dst_hbm.at[c, s], dst_v)
    pltpu.sync_copy(w_hbm.at[c, s], w_v)

    zero = jnp.zeros((16,), jnp.float32)

    def zb(i, carry):
        rows_v[i // 8, pl.ds((i % 8) * 16, 16)] = zero
        return carry

    lax.fori_loop(0, CH * 8, zb, 0)
    r0 = s * RPT
    for k in range(RPT // CH):
        pltpu.sync_copy(rows_v, accum.at[pl.ds(r0 + CH * k, CH), :])
    plsc.subcore_barrier()

    def chunk(j, carry):
        pltpu.async_copy(h_hbm.at[src_v.at[j]], rows_v, sem).wait()

        def grp(g, carry2):
            base = g * 16
            wrow = w_v[j, pl.ds(base, 16)]
            for e in range(16):
                wv = jnp.full((16,), wrow[e])
                for f in range(8):
                    sl = (base + e, pl.ds(16 * f, 16))
                    rows_v[sl] = rows_v[sl] * wv
            return carry2

        lax.fori_loop(0, 8, grp, 0)

        pltpu.sync_copy(rows_v, accum.at[dst_v.at[j]], add=True)
        return carry

    lax.fori_loop(0, NCH, chunk, 0)
    plsc.subcore_barrier()

    @pl.when(s < SUB - 1)
    def _():
        pltpu.sync_copy(
            accum.at[pl.ds(r0, RPT), :],
            out_hbm.at[c, pl.ds(r0, RPT), :],
        )

    @pl.when(s == SUB - 1)
    def _():
        last = N - (SUB - 1) * RPT
        pltpu.sync_copy(
            accum.at[pl.ds((SUB - 1) * RPT, last), :],
            out_hbm.at[c, pl.ds((SUB - 1) * RPT, last), :],
        )


_conv = functools.partial(
    pl.kernel,
    out_type=jax.ShapeDtypeStruct((CORES, N, D), jnp.float32),
    mesh=plsc.VectorSubcoreMesh(core_axis_name="c", subcore_axis_name="s"),
    scratch_types=[
        pltpu.VMEM((NCH, CH), jnp.int32),
        pltpu.VMEM((NCH, CH), jnp.int32),
        pltpu.VMEM((NCH, CH), jnp.float32),
        pltpu.VMEM((CH, D), jnp.float32),
        pltpu.VMEM_SHARED((NACC, D), jnp.float32),
        pltpu.SemaphoreType.DMA,
    ],
)(_conv_body)


def _prep_indices(edge_index, edge_weight):
    src = edge_index[0].astype(jnp.int32)
    dst = edge_index[1].astype(jnp.int32)
    w = edge_weight.astype(jnp.float32)
    pad = EPAD - E
    src_g = jnp.pad(src, (0, pad)).reshape(CORES, SUB, NCH, CH)
    dst_g = jnp.pad(dst, (0, pad)).reshape(CORES, SUB, NCH, CH)
    w_g = jnp.pad(w, (0, pad)).reshape(CORES, SUB, NCH, CH)
    return src_g, dst_g, w_g


def kernel(x, edge_index, edge_weight, W1, b1, W2, b2):
    src_g, dst_g, w_g = _prep_indices(edge_index, edge_weight)
    b1r = b1.reshape(1, D)
    b2r = b2.reshape(1, D)

    h = _matmul1(x, W1, b1r)
    parts = _conv(h, src_g, dst_g, w_g)
    h = _matmul2(parts, W2, b2r)
    parts = _conv(h, src_g, dst_g, w_g)
    return _add_parts(parts)
